# Initial kernel scaffold; baseline (speedup 1.0000x reference)
import jax
import jax.numpy as jnp
from jax.experimental import pallas as pl


def kernel(x, edge_index, edge_attr, Wl1, Wr1, b1, Wl2, Wr2, b2, Wc1, bc1, Wc2, bc2):
    raise NotImplementedError("write your pallas kernel here")


# speedup vs baseline: 3.5046x; 3.5046x over previous
import functools

import jax
import jax.numpy as jnp
from jax import lax
from jax.experimental import pallas as pl
from jax.experimental.pallas import tpu as pltpu
from jax.experimental.pallas import tpu_sc as plsc

N = 10000
E = 320000
D = 128
H = 128

NC = 2
NS = 16
NW = NC * NS
EPW = E // NW
CH = 80
NCH = EPW // CH
RPT = 632
RPT_LAST = N - (NS - 1) * RPT

_mesh = plsc.VectorSubcoreMesh(
    core_axis_name="c", subcore_axis_name="s", num_cores=NC, num_subcores=NS)


def _make_seg(with_count):
    feat_out = jax.ShapeDtypeStruct((NC * N, D), jnp.float32)
    scratch = [
        pltpu.VMEM((CH,), jnp.int32),
        pltpu.VMEM((CH,), jnp.int32),
        pltpu.VMEM((CH, D), jnp.float32),
        pltpu.VMEM_SHARED((N, D), jnp.float32),
        pltpu.SemaphoreType.DMA,
    ]
    if with_count:
        out_type = [feat_out, jax.ShapeDtypeStruct((NW, N), jnp.float32)]
        scratch.append(pltpu.VMEM((N,), jnp.float32))
    else:
        out_type = feat_out

    def body(u, src, dst, zfeat, zn, *rest):
        if with_count:
            out, outc, sbuf, dbuf, gbuf, acc, sem, cntb = rest
        else:
            out, sbuf, dbuf, gbuf, acc, sem = rest
        c = lax.axis_index("c")
        s = lax.axis_index("s")
        wid = s * NC + c
        r0 = s * RPT

        @pl.when(s < NS - 1)
        def _():
            pltpu.sync_copy(zfeat, acc.at[pl.ds(r0, RPT)])

        @pl.when(s == NS - 1)
        def _():
            pltpu.sync_copy(zfeat.at[pl.ds(0, RPT_LAST)],
                            acc.at[pl.ds(r0, RPT_LAST)])

        if with_count:
            pltpu.sync_copy(zn, cntb)
        ones = jnp.ones((16,), jnp.float32)
        plsc.subcore_barrier()
        base = wid * EPW

        def chunk(j, carry):
            gb = base + j * CH
            pltpu.sync_copy(src.at[pl.ds(gb, CH)], sbuf)
            pltpu.sync_copy(dst.at[pl.ds(gb, CH)], dbuf)
            pltpu.async_copy(u.at[sbuf], gbuf, sem).wait()
            pltpu.sync_copy(gbuf, acc.at[dbuf], add=True)
            if with_count:
                for g in range(CH // 16):
                    idx = dbuf[pl.ds(g * 16, 16)]
                    plsc.addupdate_scatter(cntb, [idx], ones)
            return carry

        lax.fori_loop(0, NCH, chunk, 0)
        plsc.subcore_barrier()

        @pl.when(s < NS - 1)
        def _():
            pltpu.sync_copy(acc.at[pl.ds(r0, RPT)],
                            out.at[pl.ds(c * N + r0, RPT)])

        @pl.when(s == NS - 1)
        def _():
            pltpu.sync_copy(acc.at[pl.ds(r0, RPT_LAST)],
                            out.at[pl.ds(c * N + r0, RPT_LAST)])

        if with_count:
            pltpu.sync_copy(cntb, outc.at[wid])

    return pl.kernel(
        body, out_type=out_type, mesh=_mesh,
        compiler_params=pltpu.CompilerParams(needs_layout_passes=False),
        scratch_types=scratch)


_seg_cnt = _make_seg(True)
_seg = _make_seg(False)


def _edge_body(a_hbm, b_hbm, src, dst, attr8, w3p, wc2p, bc2p, out,
               sbuf, dbuf, abuf, ta, tb, obuf, wbuf, c2buf, bbuf, sem, sem2):
    c = lax.axis_index("c")
    s = lax.axis_index("s")
    wid = s * NC + c
    pltpu.sync_copy(w3p, wbuf)
    pltpu.sync_copy(wc2p, c2buf)
    pltpu.sync_copy(bc2p, bbuf)
    w = [[wbuf[k, pl.ds(16 * v, 16)] for v in range(8)] for k in range(3)]
    c2 = [c2buf[pl.ds(16 * v, 16)] for v in range(8)]
    bc2s = bbuf[pl.ds(0, 16)][0]
    lane = lax.iota(jnp.int32, 16)
    base = wid * EPW

    def chunk(j, carry):
        gb = base + j * CH
        pltpu.sync_copy(src.at[pl.ds(gb, CH)], sbuf)
        pltpu.sync_copy(dst.at[pl.ds(gb, CH)], dbuf)
        pltpu.sync_copy(attr8.at[pl.ds(gb, CH)], abuf)
        cp1 = pltpu.async_copy(a_hbm.at[sbuf], ta, sem)
        cp2 = pltpu.async_copy(b_hbm.at[dbuf], tb, sem2)
        cp1.wait()
        cp2.wait()

        def group(g, cy):
            def edge(i, res):
                e = g * 16 + i
                av = abuf[e, pl.ds(0, 16)]
                a0 = av[0]
                a1 = av[1]
                a2 = av[2]
                part = None
                for v in range(8):
                    sl = pl.ds(16 * v, 16)
                    z = ta[e, sl] + tb[e, sl]
                    z = z + a0 * w[0][v] + a1 * w[1][v] + a2 * w[2][v]
                    z = jnp.maximum(z, 0.0)
                    part = z * c2[v] if part is None else part + z * c2[v]
                sc = plsc.cumsum(part)[15] + bc2s
                return jnp.where(lane == i, sc, res)

            res = lax.fori_loop(0, 16, edge, jnp.zeros((16,), jnp.float32))
            obuf[pl.ds(g * 16, 16)] = res
            return cy

        lax.fori_loop(0, CH // 16, group, 0)
        pltpu.sync_copy(obuf, out.at[pl.ds(gb, CH)])
        return carry

    lax.fori_loop(0, NCH, chunk, 0)


_edge = pl.kernel(
    _edge_body,
    out_type=jax.ShapeDtypeStruct((E,), jnp.float32),
    mesh=_mesh,
    compiler_params=pltpu.CompilerParams(needs_layout_passes=False),
    scratch_types=[
        pltpu.VMEM((CH,), jnp.int32),
        pltpu.VMEM((CH,), jnp.int32),
        pltpu.VMEM((CH, 16), jnp.float32),
        pltpu.VMEM((CH, D), jnp.float32),
        pltpu.VMEM((CH, D), jnp.float32),
        pltpu.VMEM((CH,), jnp.float32),
        pltpu.VMEM((3, D), jnp.float32),
        pltpu.VMEM((D,), jnp.float32),
        pltpu.VMEM((16,), jnp.float32),
        pltpu.SemaphoreType.DMA,
        pltpu.SemaphoreType.DMA,
    ],
)


_BR = 1000
_GRID = N // _BR


def _rowspec(cols):
    return pl.BlockSpec((_BR, cols), lambda i: (i, 0))


def _wspec(r, cols):
    return pl.BlockSpec((r, cols), lambda i: (0, 0))


def _mm_body(x_ref, w_ref, o_ref):
    o_ref[...] = jnp.dot(x_ref[...], w_ref[...],
                         preferred_element_type=jnp.float32)


_mm = pl.pallas_call(
    _mm_body,
    grid=(_GRID,),
    in_specs=[_rowspec(D), _wspec(D, H)],
    out_specs=_rowspec(H),
    out_shape=jax.ShapeDtypeStruct((N, H), jnp.float32),
)


def _cntred_body(cp_ref, o_ref):
    o_ref[...] = jnp.maximum(jnp.sum(cp_ref[...], axis=0, keepdims=True), 1.0)


_cntred = pl.pallas_call(
    _cntred_body,
    grid=(1,),
    in_specs=[pl.BlockSpec((NW, N), lambda i: (0, 0))],
    out_specs=pl.BlockSpec((1, N), lambda i: (0, 0)),
    out_shape=jax.ShapeDtypeStruct((1, N), jnp.float32),
)


def _layer1_body(p0, p1, cp, x, wr, b, wl2, h_ref, u2_ref):
    cnt = cp[...]
    mean = (p0[...] + p1[...]) / cnt
    h = jnp.maximum(mean + jnp.dot(x[...], wr[...],
                                   preferred_element_type=jnp.float32)
                    + b[...], 0.0)
    h_ref[...] = h
    u2_ref[...] = jnp.dot(h, wl2[...], preferred_element_type=jnp.float32)


_layer1 = pl.pallas_call(
    _layer1_body,
    grid=(_GRID,),
    in_specs=[_rowspec(D), _rowspec(D), _rowspec(1),
              _rowspec(D), _wspec(D, H), _wspec(1, H), _wspec(H, H)],
    out_specs=[_rowspec(H), _rowspec(H)],
    out_shape=[jax.ShapeDtypeStruct((N, H), jnp.float32),
               jax.ShapeDtypeStruct((N, H), jnp.float32)],
)


def _layer2_body(q0, q1, cp, h, wr, b, wa, ba, wb, a_ref, b_ref):
    cnt = cp[...]
    h2 = ((q0[...] + q1[...]) / cnt
          + jnp.dot(h[...], wr[...], preferred_element_type=jnp.float32)
          + b[...])
    a_ref[...] = jnp.dot(h2, wa[...],
                         preferred_element_type=jnp.float32) + ba[...]
    b_ref[...] = jnp.dot(h2, wb[...], preferred_element_type=jnp.float32)


_layer2 = pl.pallas_call(
    _layer2_body,
    grid=(_GRID,),
    in_specs=[_rowspec(H), _rowspec(H), _rowspec(1),
              _rowspec(H), _wspec(H, H), _wspec(1, H), _wspec(H, H),
              _wspec(1, H), _wspec(H, H)],
    out_specs=[_rowspec(H), _rowspec(H)],
    out_shape=[jax.ShapeDtypeStruct((N, H), jnp.float32),
               jax.ShapeDtypeStruct((N, H), jnp.float32)],
)


def kernel(x, edge_index, edge_attr, Wl1, Wr1, b1, Wl2, Wr2, b2,
           Wc1, bc1, Wc2, bc2):
    src = edge_index[0].astype(jnp.int32)
    dst = edge_index[1].astype(jnp.int32)
    attr8 = jnp.concatenate(
        [edge_attr, jnp.zeros((E, 13), jnp.float32)], axis=1)
    zfeat = jnp.zeros((RPT, D), jnp.float32)
    zn = jnp.zeros((N,), jnp.float32)

    u1 = _mm(x, Wl1)
    p, cw = _seg_cnt(u1, src, dst, zfeat, zn)
    cnt1 = _cntred(cw).reshape(N, 1)
    h, u2 = _layer1(p[:N], p[N:], cnt1, x, Wr1, b1.reshape(1, H), Wl2)

    q = _seg(u2, src, dst, zfeat, zn)
    a, b = _layer2(q[:N], q[N:], cnt1, h, Wr2, b2.reshape(1, H),
                   Wc1[:H], bc1.reshape(1, H), Wc1[H:2 * H])

    w3 = Wc1[2 * H:]
    wc2 = Wc2.reshape(H)
    bc2p = jnp.concatenate([bc2, jnp.zeros((15,), jnp.float32)])
    return _edge(a, b, src, dst, attr8, w3, wc2, bc2p)


# speedup vs baseline: 4.9409x; 1.4098x over previous
import functools

import jax
import jax.numpy as jnp
from jax import lax
from jax.experimental import pallas as pl
from jax.experimental.pallas import tpu as pltpu
from jax.experimental.pallas import tpu_sc as plsc

N = 10000
E = 320000
D = 128
H = 128

NC = 2
NS = 16
NW = NC * NS
EPW = E // NW
CH = 80
NCH = EPW // CH
RPT = 632
RPT_LAST = N - (NS - 1) * RPT

_mesh = plsc.VectorSubcoreMesh(
    core_axis_name="c", subcore_axis_name="s", num_cores=NC, num_subcores=NS)


def _make_seg(with_count):
    feat_out = jax.ShapeDtypeStruct((NC * N, D), jnp.float32)
    scratch = [
        pltpu.VMEM((2, CH), jnp.int32),
        pltpu.VMEM((2, CH), jnp.int32),
        pltpu.VMEM((2, CH, D), jnp.float32),
        pltpu.VMEM_SHARED((N, D), jnp.float32),
        pltpu.SemaphoreType.DMA,
        pltpu.SemaphoreType.DMA,
    ]
    if with_count:
        out_type = [feat_out, jax.ShapeDtypeStruct((NW, N), jnp.float32)]
        scratch.append(pltpu.VMEM((N,), jnp.float32))
    else:
        out_type = feat_out

    def body(u, src, dst, zfeat, zn, *rest):
        if with_count:
            out, outc, sbuf, dbuf, gbuf, acc, sem0, sem1, cntb = rest
        else:
            out, sbuf, dbuf, gbuf, acc, sem0, sem1 = rest
        sems = (sem0, sem1)
        c = lax.axis_index("c")
        s = lax.axis_index("s")
        wid = s * NC + c
        r0 = s * RPT

        @pl.when(s < NS - 1)
        def _():
            pltpu.sync_copy(zfeat, acc.at[pl.ds(r0, RPT)])

        @pl.when(s == NS - 1)
        def _():
            pltpu.sync_copy(zfeat.at[pl.ds(0, RPT_LAST)],
                            acc.at[pl.ds(r0, RPT_LAST)])

        if with_count:
            pltpu.sync_copy(zn, cntb)
        ones = jnp.ones((16,), jnp.float32)
        plsc.subcore_barrier()
        base = wid * EPW

        def prefetch(j, slot):
            gb = base + j * CH
            pltpu.sync_copy(src.at[pl.ds(gb, CH)], sbuf.at[slot])
            pltpu.sync_copy(dst.at[pl.ds(gb, CH)], dbuf.at[slot])
            pltpu.async_copy(u.at[sbuf.at[slot]], gbuf.at[slot], sems[slot])

        def consume(slot):
            pltpu.make_async_copy(u.at[sbuf.at[slot]], gbuf.at[slot],
                                  sems[slot]).wait()
            pltpu.sync_copy(gbuf.at[slot], acc.at[dbuf.at[slot]], add=True)
            if with_count:
                for g in range(CH // 16):
                    idx = dbuf[slot, pl.ds(g * 16, 16)]
                    plsc.addupdate_scatter(cntb, [idx], ones)

        prefetch(0, 0)

        def pair(jj, carry):
            j = jj * 2
            prefetch(j + 1, 1)
            consume(0)
            prefetch(j + 2, 0)
            consume(1)
            return carry

        lax.fori_loop(0, (NCH - 1) // 2, pair, 0)
        consume(0)
        plsc.subcore_barrier()

        @pl.when(s < NS - 1)
        def _():
            pltpu.sync_copy(acc.at[pl.ds(r0, RPT)],
                            out.at[pl.ds(c * N + r0, RPT)])

        @pl.when(s == NS - 1)
        def _():
            pltpu.sync_copy(acc.at[pl.ds(r0, RPT_LAST)],
                            out.at[pl.ds(c * N + r0, RPT_LAST)])

        if with_count:
            pltpu.sync_copy(cntb, outc.at[wid])

    return pl.kernel(
        body, out_type=out_type, mesh=_mesh,
        compiler_params=pltpu.CompilerParams(needs_layout_passes=False),
        scratch_types=scratch)


_seg_cnt = _make_seg(True)
_seg = _make_seg(False)


def _edge_body(a_hbm, b_hbm, src, dst, attr8, w3p, wc2p, bc2p, out,
               sbuf, dbuf, abuf, ta, tb, obuf, wbuf, c2buf, bbuf,
               semA0, semA1, semB0, semB1):
    semA = (semA0, semA1)
    semB = (semB0, semB1)
    c = lax.axis_index("c")
    s = lax.axis_index("s")
    wid = s * NC + c
    pltpu.sync_copy(w3p, wbuf)
    pltpu.sync_copy(wc2p, c2buf)
    pltpu.sync_copy(bc2p, bbuf)
    w = [[wbuf[k, pl.ds(16 * v, 16)] for v in range(8)] for k in range(3)]
    c2 = [c2buf[pl.ds(16 * v, 16)] for v in range(8)]
    bc2s = bbuf[pl.ds(0, 16)][0]
    lane = lax.iota(jnp.int32, 16)
    base = wid * EPW

    def prefetch(j, slot):
        gb = base + j * CH
        pltpu.sync_copy(src.at[pl.ds(gb, CH)], sbuf.at[slot])
        pltpu.sync_copy(dst.at[pl.ds(gb, CH)], dbuf.at[slot])
        pltpu.sync_copy(attr8.at[pl.ds(gb, CH)], abuf.at[slot])
        pltpu.async_copy(a_hbm.at[sbuf.at[slot]], ta.at[slot], semA[slot])
        pltpu.async_copy(b_hbm.at[dbuf.at[slot]], tb.at[slot], semB[slot])

    def wait_gathers(slot):
        pltpu.make_async_copy(a_hbm.at[sbuf.at[slot]], ta.at[slot],
                              semA[slot]).wait()
        pltpu.make_async_copy(b_hbm.at[dbuf.at[slot]], tb.at[slot],
                              semB[slot]).wait()

    def compute(j, slot):
        gb = base + j * CH

        def group(g, cy):
            def edge(i, res):
                e = g * 16 + i
                av = abuf[slot, e, pl.ds(0, 16)]
                a0 = av[0]
                a1 = av[1]
                a2 = av[2]
                part = None
                for v in range(8):
                    sl = pl.ds(16 * v, 16)
                    z = ta[slot, e, sl] + tb[slot, e, sl]
                    z = z + a0 * w[0][v] + a1 * w[1][v] + a2 * w[2][v]
                    z = jnp.maximum(z, 0.0)
                    part = z * c2[v] if part is None else part + z * c2[v]
                sc = plsc.cumsum(part)[15] + bc2s
                return jnp.where(lane == i, sc, res)

            res = lax.fori_loop(0, 16, edge, jnp.zeros((16,), jnp.float32))
            obuf[slot, pl.ds(g * 16, 16)] = res
            return cy

        lax.fori_loop(0, CH // 16, group, 0)
        pltpu.sync_copy(obuf.at[slot], out.at[pl.ds(gb, CH)])

    prefetch(0, 0)

    def pair(jj, carry):
        j = jj * 2
        wait_gathers(0)
        prefetch(j + 1, 1)
        compute(j, 0)
        wait_gathers(1)
        prefetch(j + 2, 0)
        compute(j + 1, 1)
        return carry

    lax.fori_loop(0, (NCH - 1) // 2, pair, 0)
    wait_gathers(0)
    compute(NCH - 1, 0)


_edge = pl.kernel(
    _edge_body,
    out_type=jax.ShapeDtypeStruct((E,), jnp.float32),
    mesh=_mesh,
    compiler_params=pltpu.CompilerParams(needs_layout_passes=False),
    scratch_types=[
        pltpu.VMEM((2, CH), jnp.int32),
        pltpu.VMEM((2, CH), jnp.int32),
        pltpu.VMEM((2, CH, 16), jnp.float32),
        pltpu.VMEM((2, CH, D), jnp.float32),
        pltpu.VMEM((2, CH, D), jnp.float32),
        pltpu.VMEM((2, CH), jnp.float32),
        pltpu.VMEM((3, D), jnp.float32),
        pltpu.VMEM((D,), jnp.float32),
        pltpu.VMEM((16,), jnp.float32),
        pltpu.SemaphoreType.DMA,
        pltpu.SemaphoreType.DMA,
        pltpu.SemaphoreType.DMA,
        pltpu.SemaphoreType.DMA,
    ],
)


_BR = 1000
_GRID = N // _BR


def _rowspec(cols):
    return pl.BlockSpec((_BR, cols), lambda i: (i, 0))


def _wspec(r, cols):
    return pl.BlockSpec((r, cols), lambda i: (0, 0))


def _mm_body(x_ref, w_ref, o_ref):
    o_ref[...] = jnp.dot(x_ref[...], w_ref[...],
                         preferred_element_type=jnp.float32)


_mm = pl.pallas_call(
    _mm_body,
    grid=(_GRID,),
    in_specs=[_rowspec(D), _wspec(D, H)],
    out_specs=_rowspec(H),
    out_shape=jax.ShapeDtypeStruct((N, H), jnp.float32),
)


def _cntred_body(cp_ref, o_ref):
    o_ref[...] = jnp.maximum(jnp.sum(cp_ref[...], axis=0, keepdims=True), 1.0)


_cntred = pl.pallas_call(
    _cntred_body,
    grid=(1,),
    in_specs=[pl.BlockSpec((NW, N), lambda i: (0, 0))],
    out_specs=pl.BlockSpec((1, N), lambda i: (0, 0)),
    out_shape=jax.ShapeDtypeStruct((1, N), jnp.float32),
)


def _layer1_body(p0, p1, cp, x, wr, b, wl2, h_ref, u2_ref):
    cnt = cp[...]
    mean = (p0[...] + p1[...]) / cnt
    h = jnp.maximum(mean + jnp.dot(x[...], wr[...],
                                   preferred_element_type=jnp.float32)
                    + b[...], 0.0)
    h_ref[...] = h
    u2_ref[...] = jnp.dot(h, wl2[...], preferred_element_type=jnp.float32)


_layer1 = pl.pallas_call(
    _layer1_body,
    grid=(_GRID,),
    in_specs=[_rowspec(D), _rowspec(D), _rowspec(1),
              _rowspec(D), _wspec(D, H), _wspec(1, H), _wspec(H, H)],
    out_specs=[_rowspec(H), _rowspec(H)],
    out_shape=[jax.ShapeDtypeStruct((N, H), jnp.float32),
               jax.ShapeDtypeStruct((N, H), jnp.float32)],
)


def _layer2_body(q0, q1, cp, h, wr, b, wa, ba, wb, a_ref, b_ref):
    cnt = cp[...]
    h2 = ((q0[...] + q1[...]) / cnt
          + jnp.dot(h[...], wr[...], preferred_element_type=jnp.float32)
          + b[...])
    a_ref[...] = jnp.dot(h2, wa[...],
                         preferred_element_type=jnp.float32) + ba[...]
    b_ref[...] = jnp.dot(h2, wb[...], preferred_element_type=jnp.float32)


_layer2 = pl.pallas_call(
    _layer2_body,
    grid=(_GRID,),
    in_specs=[_rowspec(H), _rowspec(H), _rowspec(1),
              _rowspec(H), _wspec(H, H), _wspec(1, H), _wspec(H, H),
              _wspec(1, H), _wspec(H, H)],
    out_specs=[_rowspec(H), _rowspec(H)],
    out_shape=[jax.ShapeDtypeStruct((N, H), jnp.float32),
               jax.ShapeDtypeStruct((N, H), jnp.float32)],
)


def kernel(x, edge_index, edge_attr, Wl1, Wr1, b1, Wl2, Wr2, b2,
           Wc1, bc1, Wc2, bc2):
    src = edge_index[0].astype(jnp.int32)
    dst = edge_index[1].astype(jnp.int32)
    attr8 = jnp.concatenate(
        [edge_attr, jnp.zeros((E, 13), jnp.float32)], axis=1)
    zfeat = jnp.zeros((RPT, D), jnp.float32)
    zn = jnp.zeros((N,), jnp.float32)

    u1 = _mm(x, Wl1)
    p, cw = _seg_cnt(u1, src, dst, zfeat, zn)
    cnt1 = _cntred(cw).reshape(N, 1)
    h, u2 = _layer1(p[:N], p[N:], cnt1, x, Wr1, b1.reshape(1, H), Wl2)

    q = _seg(u2, src, dst, zfeat, zn)
    a, b = _layer2(q[:N], q[N:], cnt1, h, Wr2, b2.reshape(1, H),
                   Wc1[:H], bc1.reshape(1, H), Wc1[H:2 * H])

    w3 = Wc1[2 * H:]
    wc2 = Wc2.reshape(H)
    bc2p = jnp.concatenate([bc2, jnp.zeros((15,), jnp.float32)])
    return _edge(a, b, src, dst, attr8, w3, wc2, bc2p)


# speedup vs baseline: 5.4079x; 1.0945x over previous
import functools

import jax
import jax.numpy as jnp
from jax import lax
from jax.experimental import pallas as pl
from jax.experimental.pallas import tpu as pltpu
from jax.experimental.pallas import tpu_sc as plsc

N = 10000
E = 320000
D = 128
H = 128

NC = 2
NS = 16
NW = NC * NS
EPW = E // NW
CH = 80
NCH = EPW // CH
RPT = 632
RPT_LAST = N - (NS - 1) * RPT

_mesh = plsc.VectorSubcoreMesh(
    core_axis_name="c", subcore_axis_name="s", num_cores=NC, num_subcores=NS)


def _make_seg(with_count):
    feat_out = jax.ShapeDtypeStruct((NC * N, D), jnp.float32)
    scratch = [
        pltpu.VMEM((2, CH), jnp.int32),
        pltpu.VMEM((2, CH), jnp.int32),
        pltpu.VMEM((2, CH, D), jnp.float32),
        pltpu.VMEM_SHARED((N, D), jnp.float32),
        pltpu.SemaphoreType.DMA,
        pltpu.SemaphoreType.DMA,
    ]
    if with_count:
        out_type = [feat_out, jax.ShapeDtypeStruct((NW, N), jnp.float32)]
        scratch.append(pltpu.VMEM((N,), jnp.float32))
    else:
        out_type = feat_out

    def body(u, src, dst, zfeat, zn, *rest):
        if with_count:
            out, outc, sbuf, dbuf, gbuf, acc, sem0, sem1, cntb = rest
        else:
            out, sbuf, dbuf, gbuf, acc, sem0, sem1 = rest
        sems = (sem0, sem1)
        c = lax.axis_index("c")
        s = lax.axis_index("s")
        wid = s * NC + c
        r0 = s * RPT

        @pl.when(s < NS - 1)
        def _():
            pltpu.sync_copy(zfeat, acc.at[pl.ds(r0, RPT)])

        @pl.when(s == NS - 1)
        def _():
            pltpu.sync_copy(zfeat.at[pl.ds(0, RPT_LAST)],
                            acc.at[pl.ds(r0, RPT_LAST)])

        if with_count:
            pltpu.sync_copy(zn, cntb)
        ones = jnp.ones((16,), jnp.float32)
        plsc.subcore_barrier()

        def prefetch(j, slot):
            pltpu.sync_copy(src.at[wid, j], sbuf.at[slot])
            pltpu.sync_copy(dst.at[wid, j], dbuf.at[slot])
            pltpu.async_copy(u.at[sbuf.at[slot]], gbuf.at[slot], sems[slot])

        def consume(j, slot):
            pltpu.make_async_copy(u.at[sbuf.at[slot]], gbuf.at[slot],
                                  sems[slot]).wait()
            pltpu.sync_copy(gbuf.at[slot], acc.at[dbuf.at[slot]], add=True)
            if with_count:
                for g in range(CH // 16):
                    idx = dbuf[slot, pl.ds(g * 16, 16)]
                    plsc.addupdate_scatter(cntb, [idx], ones)

        prefetch(0, 0)

        def pair(jj, carry):
            j = jj * 2
            prefetch(j + 1, 1)
            consume(j, 0)
            prefetch(j + 2, 0)
            consume(j + 1, 1)
            return carry

        lax.fori_loop(0, (NCH - 1) // 2, pair, 0)
        consume(NCH - 1, 0)
        plsc.subcore_barrier()

        @pl.when(s < NS - 1)
        def _():
            pltpu.sync_copy(acc.at[pl.ds(r0, RPT)],
                            out.at[pl.ds(c * N + r0, RPT)])

        @pl.when(s == NS - 1)
        def _():
            pltpu.sync_copy(acc.at[pl.ds(r0, RPT_LAST)],
                            out.at[pl.ds(c * N + r0, RPT_LAST)])

        if with_count:
            pltpu.sync_copy(cntb, outc.at[wid])

    return pl.kernel(
        body, out_type=out_type, mesh=_mesh,
        compiler_params=pltpu.CompilerParams(needs_layout_passes=False),
        scratch_types=scratch)


_seg_cnt = _make_seg(True)
_seg = _make_seg(False)


def _edge_body(a_hbm, b_hbm, src, dst, attr8, w3p, wc2p, bc2p, out,
               srcv, dstv, abuf, ta, tb, obuf, wbuf, c2buf, bbuf,
               semA0, semA1, semB0, semB1):
    semA = (semA0, semA1)
    semB = (semB0, semB1)
    c = lax.axis_index("c")
    s = lax.axis_index("s")
    wid = s * NC + c
    pltpu.sync_copy(w3p, wbuf)
    pltpu.sync_copy(wc2p, c2buf)
    pltpu.sync_copy(bc2p, bbuf)
    pltpu.sync_copy(src.at[wid], srcv)
    pltpu.sync_copy(dst.at[wid], dstv)
    w = [[wbuf[k, pl.ds(16 * v, 16)] for v in range(8)] for k in range(3)]
    c2 = [c2buf[pl.ds(16 * v, 16)] for v in range(8)]
    bc2s = bbuf[pl.ds(0, 16)][0]
    lane = lax.iota(jnp.int32, 16)
    base = wid * EPW

    def prefetch(j, slot):
        gb = base + j * CH
        pltpu.sync_copy(attr8.at[pl.ds(gb, CH)], abuf.at[slot])
        pltpu.async_copy(a_hbm.at[srcv.at[j]], ta.at[slot], semA[slot])
        pltpu.async_copy(b_hbm.at[dstv.at[j]], tb.at[slot], semB[slot])

    def wait_gathers(j, slot):
        pltpu.make_async_copy(a_hbm.at[srcv.at[j]], ta.at[slot],
                              semA[slot]).wait()
        pltpu.make_async_copy(b_hbm.at[dstv.at[j]], tb.at[slot],
                              semB[slot]).wait()

    def compute(j, slot):
        def group(g, cy):
            res = jnp.zeros((16,), jnp.float32)
            for i in range(16):
                av = abuf[slot, g * 16 + i, pl.ds(0, 16)]
                a0 = av[0]
                a1 = av[1]
                a2 = av[2]
                part = None
                for v in range(8):
                    sl = pl.ds(16 * v, 16)
                    z = ta[slot, g * 16 + i, sl] + tb[slot, g * 16 + i, sl]
                    z = z + a0 * w[0][v] + a1 * w[1][v] + a2 * w[2][v]
                    z = jnp.maximum(z, 0.0)
                    part = z * c2[v] if part is None else part + z * c2[v]
                sc = plsc.cumsum(part)[15] + bc2s
                res = jnp.where(lane == i, sc, res)
            obuf[pl.ds(j * CH + g * 16, 16)] = res
            return cy

        lax.fori_loop(0, CH // 16, group, 0)

    prefetch(0, 0)

    def pair(jj, carry):
        j = jj * 2
        wait_gathers(j, 0)
        prefetch(j + 1, 1)
        compute(j, 0)
        wait_gathers(j + 1, 1)
        prefetch(j + 2, 0)
        compute(j + 1, 1)
        return carry

    lax.fori_loop(0, (NCH - 1) // 2, pair, 0)
    wait_gathers(NCH - 1, 0)
    compute(NCH - 1, 0)
    pltpu.sync_copy(obuf, out.at[pl.ds(base, EPW)])


_edge = pl.kernel(
    _edge_body,
    out_type=jax.ShapeDtypeStruct((E,), jnp.float32),
    mesh=_mesh,
    compiler_params=pltpu.CompilerParams(needs_layout_passes=False),
    scratch_types=[
        pltpu.VMEM((NCH, CH), jnp.int32),
        pltpu.VMEM((NCH, CH), jnp.int32),
        pltpu.VMEM((2, CH, 16), jnp.float32),
        pltpu.VMEM((2, CH, D), jnp.float32),
        pltpu.VMEM((2, CH, D), jnp.float32),
        pltpu.VMEM((EPW,), jnp.float32),
        pltpu.VMEM((3, D), jnp.float32),
        pltpu.VMEM((D,), jnp.float32),
        pltpu.VMEM((16,), jnp.float32),
        pltpu.SemaphoreType.DMA,
        pltpu.SemaphoreType.DMA,
        pltpu.SemaphoreType.DMA,
        pltpu.SemaphoreType.DMA,
    ],
)


_BR = 1000
_GRID = N // _BR


def _rowspec(cols):
    return pl.BlockSpec((_BR, cols), lambda i: (i, 0))


def _wspec(r, cols):
    return pl.BlockSpec((r, cols), lambda i: (0, 0))


def _mm_body(x_ref, w_ref, o_ref):
    o_ref[...] = jnp.dot(x_ref[...], w_ref[...],
                         preferred_element_type=jnp.float32)


_mm = pl.pallas_call(
    _mm_body,
    grid=(_GRID,),
    in_specs=[_rowspec(D), _wspec(D, H)],
    out_specs=_rowspec(H),
    out_shape=jax.ShapeDtypeStruct((N, H), jnp.float32),
)


def _cntred_body(cp_ref, o_ref):
    o_ref[...] = jnp.maximum(jnp.sum(cp_ref[...], axis=0, keepdims=True), 1.0)


_cntred = pl.pallas_call(
    _cntred_body,
    grid=(1,),
    in_specs=[pl.BlockSpec((NW, N), lambda i: (0, 0))],
    out_specs=pl.BlockSpec((1, N), lambda i: (0, 0)),
    out_shape=jax.ShapeDtypeStruct((1, N), jnp.float32),
)


def _layer1_body(p0, p1, cp, x, wr, b, wl2, h_ref, u2_ref):
    cnt = cp[...]
    mean = (p0[...] + p1[...]) / cnt
    h = jnp.maximum(mean + jnp.dot(x[...], wr[...],
                                   preferred_element_type=jnp.float32)
                    + b[...], 0.0)
    h_ref[...] = h
    u2_ref[...] = jnp.dot(h, wl2[...], preferred_element_type=jnp.float32)


_layer1 = pl.pallas_call(
    _layer1_body,
    grid=(_GRID,),
    in_specs=[_rowspec(D), _rowspec(D), _rowspec(1),
              _rowspec(D), _wspec(D, H), _wspec(1, H), _wspec(H, H)],
    out_specs=[_rowspec(H), _rowspec(H)],
    out_shape=[jax.ShapeDtypeStruct((N, H), jnp.float32),
               jax.ShapeDtypeStruct((N, H), jnp.float32)],
)


def _layer2_body(q0, q1, cp, h, wr, b, wa, ba, wb, a_ref, b_ref):
    cnt = cp[...]
    h2 = ((q0[...] + q1[...]) / cnt
          + jnp.dot(h[...], wr[...], preferred_element_type=jnp.float32)
          + b[...])
    a_ref[...] = jnp.dot(h2, wa[...],
                         preferred_element_type=jnp.float32) + ba[...]
    b_ref[...] = jnp.dot(h2, wb[...], preferred_element_type=jnp.float32)


_layer2 = pl.pallas_call(
    _layer2_body,
    grid=(_GRID,),
    in_specs=[_rowspec(H), _rowspec(H), _rowspec(1),
              _rowspec(H), _wspec(H, H), _wspec(1, H), _wspec(H, H),
              _wspec(1, H), _wspec(H, H)],
    out_specs=[_rowspec(H), _rowspec(H)],
    out_shape=[jax.ShapeDtypeStruct((N, H), jnp.float32),
               jax.ShapeDtypeStruct((N, H), jnp.float32)],
)


def kernel(x, edge_index, edge_attr, Wl1, Wr1, b1, Wl2, Wr2, b2,
           Wc1, bc1, Wc2, bc2):
    src = edge_index[0].astype(jnp.int32).reshape(NW, NCH, CH)
    dst = edge_index[1].astype(jnp.int32).reshape(NW, NCH, CH)
    attr8 = jnp.concatenate(
        [edge_attr, jnp.zeros((E, 13), jnp.float32)], axis=1)
    zfeat = jnp.zeros((RPT, D), jnp.float32)
    zn = jnp.zeros((N,), jnp.float32)

    u1 = _mm(x, Wl1)
    p, cw = _seg_cnt(u1, src, dst, zfeat, zn)
    cnt1 = _cntred(cw).reshape(N, 1)
    h, u2 = _layer1(p[:N], p[N:], cnt1, x, Wr1, b1.reshape(1, H), Wl2)

    q = _seg(u2, src, dst, zfeat, zn)
    a, b = _layer2(q[:N], q[N:], cnt1, h, Wr2, b2.reshape(1, H),
                   Wc1[:H], bc1.reshape(1, H), Wc1[H:2 * H])

    w3 = Wc1[2 * H:]
    wc2 = Wc2.reshape(H)
    bc2p = jnp.concatenate([bc2, jnp.zeros((15,), jnp.float32)])
    return _edge(a, b, src, dst, attr8, w3, wc2, bc2p)


# speedup vs baseline: 6.4719x; 1.1968x over previous
import functools

import jax
import jax.numpy as jnp
from jax import lax
from jax.experimental import pallas as pl
from jax.experimental.pallas import tpu as pltpu
from jax.experimental.pallas import tpu_sc as plsc

N = 10000
E = 320000
D = 128
H = 128

NC = 2
NS = 16
NW = NC * NS
EPW = E // NW
CH = 80
NCH = EPW // CH
RPT = 632
RPT_LAST = N - (NS - 1) * RPT

_mesh = plsc.VectorSubcoreMesh(
    core_axis_name="c", subcore_axis_name="s", num_cores=NC, num_subcores=NS)


def _make_seg(with_count):
    feat_out = jax.ShapeDtypeStruct((NC * N, D), jnp.float32)
    nidx = 32
    scratch = [
        pltpu.VMEM((nidx, CH), jnp.int32),
        pltpu.VMEM((nidx, CH), jnp.int32),
        pltpu.VMEM((2, CH, D), jnp.float32),
        pltpu.VMEM_SHARED((N, D), jnp.float32),
        pltpu.SemaphoreType.DMA,
        pltpu.SemaphoreType.DMA,
    ]
    if with_count:
        out_type = [feat_out, jax.ShapeDtypeStruct((NW, N), jnp.float32)]
        scratch.append(pltpu.VMEM((N,), jnp.float32))
    else:
        out_type = feat_out

    def body(u, src, dst, zfeat, zn, *rest):
        if with_count:
            out, outc, srcv, dstv, gbuf, acc, sem0, sem1, cntb = rest
        else:
            out, srcv, dstv, gbuf, acc, sem0, sem1 = rest
        sems = (sem0, sem1)
        c = lax.axis_index("c")
        s = lax.axis_index("s")
        wid = s * NC + c
        r0 = s * RPT

        @pl.when(s < NS - 1)
        def _():
            pltpu.sync_copy(zfeat, acc.at[pl.ds(r0, RPT)])

        @pl.when(s == NS - 1)
        def _():
            pltpu.sync_copy(zfeat.at[pl.ds(0, RPT_LAST)],
                            acc.at[pl.ds(r0, RPT_LAST)])

        if with_count:
            pltpu.sync_copy(zn, cntb)
        ones = jnp.ones((16,), jnp.float32)
        plsc.subcore_barrier()

        def prefetch(row, slot):
            pltpu.async_copy(u.at[srcv.at[row]], gbuf.at[slot], sems[slot])

        def consume(row, slot):
            pltpu.make_async_copy(u.at[srcv.at[row]], gbuf.at[slot],
                                  sems[slot]).wait()
            pltpu.sync_copy(gbuf.at[slot], acc.at[dstv.at[row]], add=True)
            if with_count:
                for g in range(CH // 16):
                    idx = dstv[row, pl.ds(g * 16, 16)]
                    plsc.addupdate_scatter(cntb, [idx], ones)

        def run_block(c0, m):
            pltpu.sync_copy(src.at[wid].at[pl.ds(c0, m)],
                            srcv.at[pl.ds(0, m)])
            pltpu.sync_copy(dst.at[wid].at[pl.ds(c0, m)],
                            dstv.at[pl.ds(0, m)])
            prefetch(0, 0)

            def pair(jj, carry):
                r = jj * 2
                prefetch(r + 1, 1)
                consume(r, 0)
                prefetch(r + 2, 0)
                consume(r + 1, 1)
                return carry

            lax.fori_loop(0, (m - 1) // 2, pair, 0)
            if m % 2 == 1:
                consume(m - 1, 0)
            else:
                prefetch(m - 1, 1)
                consume(m - 2, 0)
                consume(m - 1, 1)

        run_block(0, 32)
        run_block(32, 32)
        run_block(64, 32)
        run_block(96, NCH - 96)
        plsc.subcore_barrier()

        @pl.when(s < NS - 1)
        def _():
            pltpu.sync_copy(acc.at[pl.ds(r0, RPT)],
                            out.at[pl.ds(c * N + r0, RPT)])

        @pl.when(s == NS - 1)
        def _():
            pltpu.sync_copy(acc.at[pl.ds(r0, RPT_LAST)],
                            out.at[pl.ds(c * N + r0, RPT_LAST)])

        if with_count:
            pltpu.sync_copy(cntb, outc.at[wid])

    return pl.kernel(
        body, out_type=out_type, mesh=_mesh,
        compiler_params=pltpu.CompilerParams(needs_layout_passes=False),
        scratch_types=scratch)


_seg_cnt = _make_seg(True)
_seg = _make_seg(False)


def _edge_body(a_hbm, b_hbm, src, dst, attr8, w3p, wc2p, bc2p, out,
               srcv, dstv, abuf, ta, tb, obuf, wbuf, c2buf, bbuf,
               semA0, semA1, semB0, semB1):
    semA = (semA0, semA1)
    semB = (semB0, semB1)
    c = lax.axis_index("c")
    s = lax.axis_index("s")
    wid = s * NC + c
    pltpu.sync_copy(w3p, wbuf)
    pltpu.sync_copy(wc2p, c2buf)
    pltpu.sync_copy(bc2p, bbuf)
    pltpu.sync_copy(src.at[wid], srcv)
    pltpu.sync_copy(dst.at[wid], dstv)
    w = [[wbuf[k, pl.ds(16 * v, 16)] for v in range(8)] for k in range(3)]
    c2 = [c2buf[pl.ds(16 * v, 16)] for v in range(8)]
    bc2s = bbuf[pl.ds(0, 16)][0]
    lane = lax.iota(jnp.int32, 16)
    base = wid * EPW

    ABLK = 25

    def prefetch(j, slot):
        pltpu.async_copy(a_hbm.at[srcv.at[j]], ta.at[slot], semA[slot])
        pltpu.async_copy(b_hbm.at[dstv.at[j]], tb.at[slot], semB[slot])

    def maybe_load_attr(j):
        @pl.when(lax.rem(j, ABLK) == 0)
        def _():
            pltpu.sync_copy(
                attr8.at[pl.ds((base + j * CH) * 16, ABLK * CH * 16)], abuf)

    def wait_gathers(j, slot):
        pltpu.make_async_copy(a_hbm.at[srcv.at[j]], ta.at[slot],
                              semA[slot]).wait()
        pltpu.make_async_copy(b_hbm.at[dstv.at[j]], tb.at[slot],
                              semB[slot]).wait()

    def compute(j, slot):
        arow = lax.rem(j, ABLK) * CH

        def group(g, cy):
            res = jnp.zeros((16,), jnp.float32)
            for i in range(16):
                av = abuf[pl.ds((arow + g * 16 + i) * 16, 16)]
                a0 = av[0]
                a1 = av[1]
                a2 = av[2]
                part = None
                for v in range(8):
                    sl = pl.ds(16 * v, 16)
                    z = ta[slot, g * 16 + i, sl] + tb[slot, g * 16 + i, sl]
                    z = z + a0 * w[0][v] + a1 * w[1][v] + a2 * w[2][v]
                    z = jnp.maximum(z, 0.0)
                    part = z * c2[v] if part is None else part + z * c2[v]
                sc = plsc.cumsum(part)[15] + bc2s
                res = jnp.where(lane == i, sc, res)
            obuf[pl.ds(j * CH + g * 16, 16)] = res
            return cy

        lax.fori_loop(0, CH // 16, group, 0)

    prefetch(0, 0)

    def pair(jj, carry):
        j = jj * 2
        wait_gathers(j, 0)
        prefetch(j + 1, 1)
        maybe_load_attr(j)
        compute(j, 0)
        wait_gathers(j + 1, 1)
        prefetch(j + 2, 0)
        maybe_load_attr(j + 1)
        compute(j + 1, 1)
        return carry

    lax.fori_loop(0, (NCH - 1) // 2, pair, 0)
    wait_gathers(NCH - 1, 0)
    compute(NCH - 1, 0)
    pltpu.sync_copy(obuf, out.at[pl.ds(base, EPW)])


_edge = pl.kernel(
    _edge_body,
    out_type=jax.ShapeDtypeStruct((E,), jnp.float32),
    mesh=_mesh,
    compiler_params=pltpu.CompilerParams(needs_layout_passes=False),
    scratch_types=[
        pltpu.VMEM((NCH, CH), jnp.int32),
        pltpu.VMEM((NCH, CH), jnp.int32),
        pltpu.VMEM((25 * CH * 16,), jnp.float32),
        pltpu.VMEM((2, CH, D), jnp.float32),
        pltpu.VMEM((2, CH, D), jnp.float32),
        pltpu.VMEM((EPW,), jnp.float32),
        pltpu.VMEM((3, D), jnp.float32),
        pltpu.VMEM((D,), jnp.float32),
        pltpu.VMEM((16,), jnp.float32),
        pltpu.SemaphoreType.DMA,
        pltpu.SemaphoreType.DMA,
        pltpu.SemaphoreType.DMA,
        pltpu.SemaphoreType.DMA,
    ],
)


_BR = 1000
_GRID = N // _BR


def _rowspec(cols):
    return pl.BlockSpec((_BR, cols), lambda i: (i, 0))


def _wspec(r, cols):
    return pl.BlockSpec((r, cols), lambda i: (0, 0))


def _mm_body(x_ref, w_ref, o_ref):
    o_ref[...] = jnp.dot(x_ref[...], w_ref[...],
                         preferred_element_type=jnp.float32)


_mm = pl.pallas_call(
    _mm_body,
    grid=(_GRID,),
    in_specs=[_rowspec(D), _wspec(D, H)],
    out_specs=_rowspec(H),
    out_shape=jax.ShapeDtypeStruct((N, H), jnp.float32),
)


def _cntred_body(cp_ref, o_ref):
    o_ref[...] = jnp.maximum(jnp.sum(cp_ref[...], axis=0, keepdims=True), 1.0)


_cntred = pl.pallas_call(
    _cntred_body,
    grid=(1,),
    in_specs=[pl.BlockSpec((NW, N), lambda i: (0, 0))],
    out_specs=pl.BlockSpec((1, N), lambda i: (0, 0)),
    out_shape=jax.ShapeDtypeStruct((1, N), jnp.float32),
)


def _layer1_body(p0, p1, cp, x, wr, b, wl2, h_ref, u2_ref):
    cnt = cp[...]
    mean = (p0[...] + p1[...]) / cnt
    h = jnp.maximum(mean + jnp.dot(x[...], wr[...],
                                   preferred_element_type=jnp.float32)
                    + b[...], 0.0)
    h_ref[...] = h
    u2_ref[...] = jnp.dot(h, wl2[...], preferred_element_type=jnp.float32)


_layer1 = pl.pallas_call(
    _layer1_body,
    grid=(_GRID,),
    in_specs=[_rowspec(D), _rowspec(D), _rowspec(1),
              _rowspec(D), _wspec(D, H), _wspec(1, H), _wspec(H, H)],
    out_specs=[_rowspec(H), _rowspec(H)],
    out_shape=[jax.ShapeDtypeStruct((N, H), jnp.float32),
               jax.ShapeDtypeStruct((N, H), jnp.float32)],
)


def _layer2_body(q0, q1, cp, h, wr, b, wa, ba, wb, a_ref, b_ref):
    cnt = cp[...]
    h2 = ((q0[...] + q1[...]) / cnt
          + jnp.dot(h[...], wr[...], preferred_element_type=jnp.float32)
          + b[...])
    a_ref[...] = jnp.dot(h2, wa[...],
                         preferred_element_type=jnp.float32) + ba[...]
    b_ref[...] = jnp.dot(h2, wb[...], preferred_element_type=jnp.float32)


_layer2 = pl.pallas_call(
    _layer2_body,
    grid=(_GRID,),
    in_specs=[_rowspec(H), _rowspec(H), _rowspec(1),
              _rowspec(H), _wspec(H, H), _wspec(1, H), _wspec(H, H),
              _wspec(1, H), _wspec(H, H)],
    out_specs=[_rowspec(H), _rowspec(H)],
    out_shape=[jax.ShapeDtypeStruct((N, H), jnp.float32),
               jax.ShapeDtypeStruct((N, H), jnp.float32)],
)


def kernel(x, edge_index, edge_attr, Wl1, Wr1, b1, Wl2, Wr2, b2,
           Wc1, bc1, Wc2, bc2):
    src = edge_index[0].astype(jnp.int32).reshape(NW, NCH, CH)
    dst = edge_index[1].astype(jnp.int32).reshape(NW, NCH, CH)
    attr8 = jnp.concatenate(
        [edge_attr, jnp.zeros((E, 13), jnp.float32)], axis=1).reshape(E * 16)
    zfeat = jnp.zeros((RPT, D), jnp.float32)
    zn = jnp.zeros((N,), jnp.float32)

    u1 = _mm(x, Wl1)
    p, cw = _seg_cnt(u1, src, dst, zfeat, zn)
    cnt1 = _cntred(cw).reshape(N, 1)
    h, u2 = _layer1(p[:N], p[N:], cnt1, x, Wr1, b1.reshape(1, H), Wl2)

    q = _seg(u2, src, dst, zfeat, zn)
    a, b = _layer2(q[:N], q[N:], cnt1, h, Wr2, b2.reshape(1, H),
                   Wc1[:H], bc1.reshape(1, H), Wc1[H:2 * H])

    w3 = Wc1[2 * H:]
    wc2 = Wc2.reshape(H)
    bc2p = jnp.concatenate([bc2, jnp.zeros((15,), jnp.float32)])
    return _edge(a, b, src, dst, attr8, w3, wc2, bc2p)
